# baseline (speedup 1.0000x reference)
import functools

import jax
import jax.numpy as jnp
from jax import lax
from jax.experimental import pallas as pl
from jax.experimental.pallas import tpu as pltpu
from jax.experimental.pallas import tpu_sc as plsc

_CHUNK = 128
_L = 16


@functools.lru_cache(maxsize=None)
def _build(B, V, D):
    info = plsc.get_sparse_core_info()
    NC, NS = info.num_cores, info.num_subcores
    NW = NC * NS
    b_per_w = B // NW
    n_chunk = b_per_w // _CHUNK
    rows_per_d = V // _L
    assert B % (NW * _CHUNK) == 0 and V % _L == 0 and D % 2 == 0
    mesh = plsc.VectorSubcoreMesh(core_axis_name="c", subcore_axis_name="s")

    @functools.partial(
        pl.kernel,
        mesh=mesh,
        out_type=jax.ShapeDtypeStruct((D, B), jnp.float32),
        scratch_types=[
            pltpu.VMEM((n_chunk, _CHUNK), jnp.int32),
            pltpu.VMEM((n_chunk, _CHUNK), jnp.int32),
            pltpu.VMEM((n_chunk, _CHUNK), jnp.int32),
            pltpu.VMEM((2, n_chunk, _CHUNK), jnp.int32),
            pltpu.VMEM((2, b_per_w, _L), jnp.float32),
            pltpu.VMEM((D, b_per_w), jnp.float32),
            pltpu.SemaphoreType.DMA,
        ],
        compiler_params=pltpu.CompilerParams(
            needs_layout_passes=False, use_tc_tiling_on_sc=False
        ),
    )
    def k(idx_hbm, table_hbm, out_hbm, idx_v, lane_v, mrow_v, ridx_v,
          gbuf_v, obuf_v, sem):
        wid = lax.axis_index("s") * NC + lax.axis_index("c")
        base = wid * b_per_w
        pltpu.sync_copy(idx_hbm.at[pl.ds(wid * n_chunk, n_chunk)], idx_v)

        for j in range(n_chunk):
            for s in range(0, _CHUNK, _L):
                v = idx_v[j, pl.ds(s, _L)]
                lane_v[j, pl.ds(s, _L)] = lax.bitwise_and(v, _L - 1)
                mrow_v[j, pl.ds(s, _L)] = lax.shift_right_logical(v, 4)

        iota = lax.iota(jnp.int32, _L)

        def fire(d):
            slot = lax.rem(d, 2)
            off = d * rows_per_d
            for j in range(n_chunk):
                for s in range(0, _CHUNK, _L):
                    ridx_v[slot, j, pl.ds(s, _L)] = (
                        mrow_v[j, pl.ds(s, _L)] + off
                    )
            for j in range(n_chunk):
                pltpu.async_copy(
                    table_hbm.at[ridx_v.at[slot, j]],
                    gbuf_v.at[slot, pl.ds(j * _CHUNK, _CHUNK), :],
                    sem,
                )

        def consume(d):
            slot = lax.rem(d, 2)
            pltpu.make_async_copy(
                table_hbm.at[pl.ds(0, b_per_w)], gbuf_v.at[slot], sem
            ).wait()
            g2d = gbuf_v.at[slot]
            for g in range(b_per_w // _L):
                rows = iota + (g * _L)
                lanes = lane_v[g // (_CHUNK // _L),
                               pl.ds((g % (_CHUNK // _L)) * _L, _L)]
                vals = plsc.load_gather(g2d, [rows, lanes])
                obuf_v[d, pl.ds(g * _L, _L)] = vals

        def body(d, carry):
            pl.when(d < D)(lambda: fire(d))
            pl.when(d > 0)(lambda: consume(d - 1))
            return carry

        lax.fori_loop(0, D + 1, body, 0)
        pltpu.sync_copy(obuf_v, out_hbm.at[:, pl.ds(base, b_per_w)])

    return k


def kernel(user_id, embedding_weight):
    B = user_id.shape[0]
    V, D = embedding_weight.shape
    idx = user_id.astype(jnp.int32).reshape(B // _CHUNK, _CHUNK)
    table2 = embedding_weight.T.reshape((D * V) // _L, _L)
    out_t = _build(B, V, D)(idx, table2)
    return out_t.T
